# baseline (speedup 1.0000x reference)
import functools

import jax
import jax.numpy as jnp
from jax import lax
from jax.experimental import pallas as pl
from jax.experimental.pallas import tpu as pltpu
from jax.experimental.pallas import tpu_sc as plsc

_S, _D, _H, _E = 2048, 1024, 2048, 8
_BM = 512
_TT = 15
_TP = _TT * _BM
_NW = 32
_BAL, _Z = 0.01, 0.001


def _router_body(x_ref, wg_ref, cw_ref, loss_ref):
    x = x_ref[...]
    wg = wg_ref[...]
    logits = jnp.dot(x, wg, preferred_element_type=jnp.float32)
    iota = lax.broadcasted_iota(jnp.int32, (_S, _E), 1)
    m1 = jnp.max(logits, axis=1, keepdims=True)
    a1 = jnp.min(jnp.where(logits == m1, iota, _E), axis=1, keepdims=True)
    lm = jnp.where(iota == a1, jnp.float32(-1e30), logits)
    m2 = jnp.max(lm, axis=1, keepdims=True)
    a2 = jnp.min(jnp.where(lm == m2, iota, _E), axis=1, keepdims=True)
    e2 = jnp.exp(m2 - m1)
    w1 = 1.0 / (1.0 + e2)
    w2 = e2 / (1.0 + e2)
    cw = jnp.where(iota == a1, w1, 0.0) + jnp.where(iota == a2, w2, 0.0)
    cw_ref[...] = cw
    lse = m1 + jnp.log(jnp.sum(jnp.exp(logits - m1), axis=1, keepdims=True))
    zloss = jnp.mean(lse * lse) * _Z
    usage = jnp.sum(cw, axis=0)
    um = jnp.mean(usage)
    ustd = jnp.sqrt(jnp.mean((usage - um) ** 2))
    loss_ref[...] = jnp.full((1, 1), ustd / um * _BAL + zloss, jnp.float32)


_router = pl.pallas_call(
    _router_body,
    out_shape=(
        jax.ShapeDtypeStruct((_S, _E), jnp.float32),
        jax.ShapeDtypeStruct((1, 1), jnp.float32),
    ),
)


_sc_mesh = plsc.VectorSubcoreMesh(
    core_axis_name="c", subcore_axis_name="s", num_cores=2, num_subcores=16)


@functools.partial(
    pl.kernel,
    out_type=(
        jax.ShapeDtypeStruct((_TP, _D), jnp.float32),
        jax.ShapeDtypeStruct((_TP, 128), jnp.float32),
    ),
    mesh=_sc_mesh,
    scratch_types=[
        pltpu.VMEM((8, 16), jnp.int32),
        pltpu.VMEM((8, 16, 128), jnp.float32),
        pltpu.VMEM((16, _D), jnp.float32),
        pltpu.VMEM((16, _D), jnp.float32),
        pltpu.VMEM((16, _D), jnp.float32),
        pltpu.VMEM((16, _D), jnp.float32),
        pltpu.SemaphoreType.DMA,
        pltpu.SemaphoreType.DMA,
        pltpu.SemaphoreType.DMA,
        pltpu.SemaphoreType.DMA,
        pltpu.SemaphoreType.DMA,
    ],
)
def _dispatch(dest_hbm, x_hbm, w16_hbm, xs_hbm, wr_hbm,
              idx2d, wbuf, xb0, xb1, xb2, xb3, s0, s1, s2, s3, sw):
    wid = lax.axis_index("s") * 2 + lax.axis_index("c")
    base = wid * 128
    tb = (wid % 16) * 128
    for j in range(8):
        pltpu.sync_copy(dest_hbm.at[pl.ds(base + j * 16, 16)], idx2d.at[j])
        pltpu.sync_copy(w16_hbm.at[pl.ds(base + j * 16, 16)], wbuf.at[j])
    xb = (xb0, xb1, xb2, xb3)
    sems = (s0, s1, s2, s3)
    hl, hs = {}, {}
    for j in range(4):
        hl[j] = pltpu.async_copy(x_hbm.at[pl.ds(tb + j * 16, 16)], xb[j], sems[j])
    for j in range(8):
        b = j % 4
        if j >= 4:
            hs[j - 4].wait()
            hl[j] = pltpu.async_copy(
                x_hbm.at[pl.ds(tb + j * 16, 16)], xb[b], sems[b])
        hl[j].wait()
        hs[j] = pltpu.async_copy(xb[b], xs_hbm.at[idx2d.at[j]], sems[b])
    hw = [pltpu.async_copy(wbuf.at[j], wr_hbm.at[idx2d.at[j]], sw)
          for j in range(8)]
    for j in range(4, 8):
        hs[j].wait()
    for h in hw:
        h.wait()


def _ffn_body(eids_ref, x_ref, w1_ref, b1_ref, w2_ref, b2_ref, wrow_ref, y_ref):
    t = pl.program_id(0)
    hh = pl.program_id(1)

    @pl.when(t < eids_ref[_TT])
    def _():
        xb = x_ref[...].astype(jnp.bfloat16)
        h = jnp.dot(xb, w1_ref[0].astype(jnp.bfloat16),
                    preferred_element_type=jnp.float32)
        h = jax.nn.gelu(h + b1_ref[0])
        part = jnp.dot(h.astype(jnp.bfloat16), w2_ref[0].astype(jnp.bfloat16),
                       preferred_element_type=jnp.float32)

        @pl.when(hh == 0)
        def _():
            y_ref[...] = part

        @pl.when(hh == 1)
        def _():
            y_ref[...] = (y_ref[...] + part + b2_ref[0]) * wrow_ref[...][:, 0:1]


_ffn = pl.pallas_call(
    _ffn_body,
    grid_spec=pltpu.PrefetchScalarGridSpec(
        num_scalar_prefetch=1,
        grid=(_TT, 2),
        in_specs=[
            pl.BlockSpec((_BM, _D), lambda t, hh, eids: (t, 0)),
            pl.BlockSpec((1, _D, _H // 2), lambda t, hh, eids: (eids[t], 0, hh)),
            pl.BlockSpec((1, 1, _H // 2), lambda t, hh, eids: (eids[t], 0, hh)),
            pl.BlockSpec((1, _H // 2, _D), lambda t, hh, eids: (eids[t], hh, 0)),
            pl.BlockSpec((1, 1, _D), lambda t, hh, eids: (eids[t], 0, 0)),
            pl.BlockSpec((_BM, 128), lambda t, hh, eids: (t, 0)),
        ],
        out_specs=pl.BlockSpec((_BM, _D), lambda t, hh, eids: (t, 0)),
    ),
    out_shape=jax.ShapeDtypeStruct((_TP, _D), jnp.float32),
)


def _shared_body(x_ref, w1_ref, b1_ref, w2_ref, b2_ref, y_ref):
    xb = x_ref[...].astype(jnp.bfloat16)
    h = jnp.dot(xb, w1_ref[...].astype(jnp.bfloat16),
                preferred_element_type=jnp.float32)
    h = jax.nn.gelu(h + b1_ref[...])
    y_ref[...] = jnp.dot(
        h.astype(jnp.bfloat16), w2_ref[...].astype(jnp.bfloat16),
        preferred_element_type=jnp.float32) + b2_ref[...]


_shared = pl.pallas_call(
    _shared_body,
    grid=(_S // _BM,),
    in_specs=[
        pl.BlockSpec((_BM, _D), lambda t: (t, 0)),
        pl.BlockSpec((_D, _H), lambda t: (0, 0)),
        pl.BlockSpec((1, _H), lambda t: (0, 0)),
        pl.BlockSpec((_H, _D), lambda t: (0, 0)),
        pl.BlockSpec((1, _D), lambda t: (0, 0)),
    ],
    out_specs=pl.BlockSpec((_BM, _D), lambda t: (t, 0)),
    out_shape=jax.ShapeDtypeStruct((_S, _D), jnp.float32),
)


@functools.partial(
    pl.kernel,
    out_type=jax.ShapeDtypeStruct((_S, _D), jnp.float32),
    mesh=plsc.VectorSubcoreMesh(
        core_axis_name="c", subcore_axis_name="s", num_cores=2, num_subcores=16),
    scratch_types=[
        pltpu.VMEM((_S // _NW,), jnp.int32),
        pltpu.VMEM((_S // _NW,), jnp.int32),
        pltpu.VMEM((16, _D), jnp.float32),
        pltpu.VMEM((16, _D), jnp.float32),
        pltpu.VMEM((16, _D), jnp.float32),
        pltpu.VMEM((16, _D), jnp.float32),
        pltpu.VMEM((16, _D), jnp.float32),
        pltpu.VMEM((16, _D), jnp.float32),
        pltpu.SemaphoreType.DMA,
        pltpu.SemaphoreType.DMA,
    ],
)
def _combine(p0_hbm, p1_hbm, ysh_hbm, ys_hbm, out_hbm,
             p0_v, p1_v, sh0, r00, r10, sh1, r01, r11, sem0, sem1):
    wid = lax.axis_index("s") * 2 + lax.axis_index("c")
    tok_per = _S // _NW
    base = wid * tok_per
    pltpu.sync_copy(p0_hbm.at[pl.ds(base, tok_per)], p0_v)
    pltpu.sync_copy(p1_hbm.at[pl.ds(base, tok_per)], p1_v)
    bufs = ((sh0, r00, r10, sem0), (sh1, r01, r11, sem1))

    def fire(j):
        sh, r0, r1, sem = bufs[j % 2]
        sl16 = pl.ds(j * 16, 16)
        return (
            pltpu.async_copy(ysh_hbm.at[pl.ds(base + j * 16, 16)], sh, sem),
            pltpu.async_copy(ys_hbm.at[p0_v.at[sl16]], r0, sem),
            pltpu.async_copy(ys_hbm.at[p1_v.at[sl16]], r1, sem),
        )

    pending = fire(0)
    for j in range(4):
        nxt = fire(j + 1) if j < 3 else None
        for c in pending:
            c.wait()
        sh, r0, r1, _ = bufs[j % 2]
        for i in range(16):
            def dstep(k, c, i=i):
                sl = pl.ds(k * 16, 16)
                sh[i, sl] = sh[i, sl] + r0[i, sl] + r1[i, sl]
                return c

            lax.fori_loop(0, _D // 16, dstep, 0)
        pltpu.sync_copy(sh, out_hbm.at[pl.ds(base + j * 16, 16)])
        pending = nxt


def kernel(x, Wg, W1, b1, W2, b2, Ws1, bs1, Ws2, bs2):
    x_flat = x.reshape(_S, _D)
    cw, loss = _router(x_flat, Wg)
    router_loss = loss[0, 0]

    iota8 = jnp.arange(_E, dtype=jnp.int32)
    a1 = jnp.argmax(cw, axis=1).astype(jnp.int32)
    cwm = jnp.where(iota8[None, :] == a1[:, None], -1.0, cw)
    a2 = jnp.argmax(cwm, axis=1).astype(jnp.int32)
    w0 = jnp.take_along_axis(cw, a1[:, None], axis=1)[:, 0]
    w1 = jnp.take_along_axis(cw, a2[:, None], axis=1)[:, 0]
    e_all = jnp.concatenate([a1, a2])
    onehot = (e_all[:, None] == iota8[None, :]).astype(jnp.int32)
    rank = jnp.sum((jnp.cumsum(onehot, axis=0) - onehot) * onehot, axis=1)
    counts = jnp.sum(onehot, axis=0)
    tile_counts = (counts + _BM - 1) // _BM
    end_tile = jnp.cumsum(tile_counts)
    start_row = (end_tile - tile_counts) * _BM
    dest = (start_row[e_all] + rank).astype(jnp.int32)
    tvec = jnp.arange(_TT, dtype=jnp.int32)
    eid = jnp.minimum(
        jnp.sum((tvec[:, None] >= end_tile[None, :]).astype(jnp.int32), axis=1),
        _E - 1)
    eids = jnp.concatenate([eid, end_tile[_E - 1:]]).astype(jnp.int32)
    w16 = jnp.broadcast_to(
        jnp.concatenate([w0, w1])[:, None], (2 * _S, 128))
    xs, wrow = _dispatch(dest, x_flat, w16)
    ys = _ffn(eids, xs, W1, b1.reshape(_E, 1, _H), W2, b2.reshape(_E, 1, _D),
              wrow)
    ysh = _shared(x_flat, Ws1, bs1.reshape(1, _H), Ws2, bs2.reshape(1, _D))
    out = _combine(dest[:_S], dest[_S:], ysh, ys)
    return out.reshape(1, _S, _D), router_loss


# speedup vs baseline: 1.2015x; 1.2015x over previous
import functools

import jax
import jax.numpy as jnp
from jax import lax
from jax.experimental import pallas as pl
from jax.experimental.pallas import tpu as pltpu
from jax.experimental.pallas import tpu_sc as plsc

_S, _D, _H, _E = 2048, 1024, 2048, 8
_BM = 512
_TT = 15
_TP = _TT * _BM
_NW = 32
_BAL, _Z = 0.01, 0.001


def _router_body(x_ref, wg_ref, idx_ref, wa_ref, wb_ref, loss_ref):
    x = x_ref[...]
    wg = wg_ref[...]
    logits = jnp.dot(x, wg, preferred_element_type=jnp.float32)
    iota = lax.broadcasted_iota(jnp.int32, (_S, _E), 1)
    m1 = jnp.max(logits, axis=1, keepdims=True)
    a1 = jnp.min(jnp.where(logits == m1, iota, _E), axis=1, keepdims=True)
    lm = jnp.where(iota == a1, jnp.float32(-1e30), logits)
    m2 = jnp.max(lm, axis=1, keepdims=True)
    a2 = jnp.min(jnp.where(lm == m2, iota, _E), axis=1, keepdims=True)
    e2 = jnp.exp(m2 - m1)
    w1 = 1.0 / (1.0 + e2)
    w2 = e2 / (1.0 + e2)
    cw = jnp.where(iota == a1, w1, 0.0) + jnp.where(iota == a2, w2, 0.0)
    idx_ref[...] = a1 * (iota == 0).astype(jnp.int32) + \
        a2 * (iota == 1).astype(jnp.int32)
    wa_ref[...] = jnp.broadcast_to(w1, (_S, 128))
    wb_ref[...] = jnp.broadcast_to(w2, (_S, 128))
    lse = m1 + jnp.log(jnp.sum(jnp.exp(logits - m1), axis=1, keepdims=True))
    zloss = jnp.mean(lse * lse) * _Z
    usage = jnp.sum(cw, axis=0)
    um = jnp.mean(usage)
    ustd = jnp.sqrt(jnp.mean((usage - um) ** 2))
    loss_ref[...] = jnp.full((1, 1), ustd / um * _BAL + zloss, jnp.float32)


_router = pl.pallas_call(
    _router_body,
    out_shape=(
        jax.ShapeDtypeStruct((_S, _E), jnp.int32),
        jax.ShapeDtypeStruct((_S, 128), jnp.float32),
        jax.ShapeDtypeStruct((_S, 128), jnp.float32),
        jax.ShapeDtypeStruct((1, 1), jnp.float32),
    ),
)


_sc_mesh = plsc.VectorSubcoreMesh(
    core_axis_name="c", subcore_axis_name="s", num_cores=2, num_subcores=16)


@functools.partial(
    pl.kernel,
    out_type=(
        jax.ShapeDtypeStruct((_TP, _D), jnp.float32),
        jax.ShapeDtypeStruct((_TP, 128), jnp.float32),
    ),
    mesh=_sc_mesh,
    scratch_types=[
        pltpu.VMEM((8, 16), jnp.int32),
        pltpu.VMEM((8, 16, 128), jnp.float32),
        pltpu.VMEM((16, _D), jnp.float32),
        pltpu.VMEM((16, _D), jnp.float32),
        pltpu.VMEM((16, _D), jnp.float32),
        pltpu.VMEM((16, _D), jnp.float32),
        pltpu.SemaphoreType.DMA,
        pltpu.SemaphoreType.DMA,
        pltpu.SemaphoreType.DMA,
        pltpu.SemaphoreType.DMA,
        pltpu.SemaphoreType.DMA,
    ],
)
def _dispatch(dest_hbm, x_hbm, wa_hbm, wb_hbm, xs_hbm, wr_hbm,
              idx2d, wbuf, xb0, xb1, xb2, xb3, s0, s1, s2, s3, sw):
    wid = lax.axis_index("s") * 2 + lax.axis_index("c")
    base = wid * 128
    tb = (wid % 16) * 128
    for j in range(8):
        pltpu.sync_copy(dest_hbm.at[pl.ds(base + j * 16, 16)], idx2d.at[j])

    @pl.when(wid < 16)
    def _():
        for j in range(8):
            pltpu.sync_copy(wa_hbm.at[pl.ds(tb + j * 16, 16)], wbuf.at[j])

    @pl.when(wid >= 16)
    def _():
        for j in range(8):
            pltpu.sync_copy(wb_hbm.at[pl.ds(tb + j * 16, 16)], wbuf.at[j])
    xb = (xb0, xb1, xb2, xb3)
    sems = (s0, s1, s2, s3)
    hl, hs = {}, {}
    for j in range(4):
        hl[j] = pltpu.async_copy(x_hbm.at[pl.ds(tb + j * 16, 16)], xb[j], sems[j])
    for j in range(8):
        b = j % 4
        if j >= 4:
            hs[j - 4].wait()
            hl[j] = pltpu.async_copy(
                x_hbm.at[pl.ds(tb + j * 16, 16)], xb[b], sems[b])
        hl[j].wait()
        hs[j] = pltpu.async_copy(xb[b], xs_hbm.at[idx2d.at[j]], sems[b])
    hw = [pltpu.async_copy(wbuf.at[j], wr_hbm.at[idx2d.at[j]], sw)
          for j in range(8)]
    for j in range(4, 8):
        hs[j].wait()
    for h in hw:
        h.wait()


def _ffn_body(eids_ref, x_ref, w1_ref, b1_ref, w2_ref, b2_ref, wrow_ref, y_ref):
    t = pl.program_id(0)

    @pl.when(t < eids_ref[_TT])
    def _():
        xb = x_ref[...].astype(jnp.bfloat16)
        h = jnp.dot(xb, w1_ref[0].astype(jnp.bfloat16),
                    preferred_element_type=jnp.float32)
        h = jax.nn.gelu(h + b1_ref[0])
        y = jnp.dot(h.astype(jnp.bfloat16), w2_ref[0].astype(jnp.bfloat16),
                    preferred_element_type=jnp.float32)
        y_ref[...] = (y + b2_ref[0]) * wrow_ref[...][:, 0:1]


_ffn = pl.pallas_call(
    _ffn_body,
    grid_spec=pltpu.PrefetchScalarGridSpec(
        num_scalar_prefetch=1,
        grid=(_TT,),
        in_specs=[
            pl.BlockSpec((_BM, _D), lambda t, eids: (t, 0)),
            pl.BlockSpec((1, _D, _H), lambda t, eids: (eids[t], 0, 0)),
            pl.BlockSpec((1, 1, _H), lambda t, eids: (eids[t], 0, 0)),
            pl.BlockSpec((1, _H, _D), lambda t, eids: (eids[t], 0, 0)),
            pl.BlockSpec((1, 1, _D), lambda t, eids: (eids[t], 0, 0)),
            pl.BlockSpec((_BM, 128), lambda t, eids: (t, 0)),
        ],
        out_specs=pl.BlockSpec((_BM, _D), lambda t, eids: (t, 0)),
    ),
    out_shape=jax.ShapeDtypeStruct((_TP, _D), jnp.float32),
)


def _shared_body(x_ref, w1_ref, b1_ref, w2_ref, b2_ref, y_ref):
    xb = x_ref[...].astype(jnp.bfloat16)
    h = jnp.dot(xb, w1_ref[...].astype(jnp.bfloat16),
                preferred_element_type=jnp.float32)
    h = jax.nn.gelu(h + b1_ref[...])
    y_ref[...] = jnp.dot(
        h.astype(jnp.bfloat16), w2_ref[...].astype(jnp.bfloat16),
        preferred_element_type=jnp.float32) + b2_ref[...]


_shared = pl.pallas_call(
    _shared_body,
    grid=(_S // _BM,),
    in_specs=[
        pl.BlockSpec((_BM, _D), lambda t: (t, 0)),
        pl.BlockSpec((_D, _H), lambda t: (0, 0)),
        pl.BlockSpec((1, _H), lambda t: (0, 0)),
        pl.BlockSpec((_H, _D), lambda t: (0, 0)),
        pl.BlockSpec((1, _D), lambda t: (0, 0)),
    ],
    out_specs=pl.BlockSpec((_BM, _D), lambda t: (t, 0)),
    out_shape=jax.ShapeDtypeStruct((_S, _D), jnp.float32),
)


@functools.partial(
    pl.kernel,
    out_type=jax.ShapeDtypeStruct((_S, _D), jnp.float32),
    mesh=plsc.VectorSubcoreMesh(
        core_axis_name="c", subcore_axis_name="s", num_cores=2, num_subcores=16),
    scratch_types=[
        pltpu.VMEM((_S // _NW,), jnp.int32),
        pltpu.VMEM((_S // _NW,), jnp.int32),
        pltpu.VMEM((16, _D), jnp.float32),
        pltpu.VMEM((16, _D), jnp.float32),
        pltpu.VMEM((16, _D), jnp.float32),
        pltpu.VMEM((16, _D), jnp.float32),
        pltpu.VMEM((16, _D), jnp.float32),
        pltpu.VMEM((16, _D), jnp.float32),
        pltpu.SemaphoreType.DMA,
        pltpu.SemaphoreType.DMA,
    ],
)
def _combine(p0_hbm, p1_hbm, ysh_hbm, ys_hbm, out_hbm,
             p0_v, p1_v, sh0, r00, r10, sh1, r01, r11, sem0, sem1):
    wid = lax.axis_index("s") * 2 + lax.axis_index("c")
    tok_per = _S // _NW
    base = wid * tok_per
    pltpu.sync_copy(p0_hbm.at[pl.ds(base, tok_per)], p0_v)
    pltpu.sync_copy(p1_hbm.at[pl.ds(base, tok_per)], p1_v)
    bufs = ((sh0, r00, r10, sem0), (sh1, r01, r11, sem1))

    def fire(j):
        sh, r0, r1, sem = bufs[j % 2]
        sl16 = pl.ds(j * 16, 16)
        return (
            pltpu.async_copy(ysh_hbm.at[pl.ds(base + j * 16, 16)], sh, sem),
            pltpu.async_copy(ys_hbm.at[p0_v.at[sl16]], r0, sem),
            pltpu.async_copy(ys_hbm.at[p1_v.at[sl16]], r1, sem),
        )

    pending = fire(0)
    for j in range(4):
        nxt = fire(j + 1) if j < 3 else None
        for c in pending:
            c.wait()
        sh, r0, r1, _ = bufs[j % 2]
        for i in range(16):
            def dstep(k, c, i=i):
                sl = pl.ds(k * 16, 16)
                sh[i, sl] = sh[i, sl] + r0[i, sl] + r1[i, sl]
                return c

            lax.fori_loop(0, _D // 16, dstep, 0)
        pltpu.sync_copy(sh, out_hbm.at[pl.ds(base + j * 16, 16)])
        pending = nxt


def kernel(x, Wg, W1, b1, W2, b2, Ws1, bs1, Ws2, bs2):
    x_flat = x.reshape(_S, _D)
    idx2, wa, wb, loss = _router(x_flat, Wg)
    router_loss = loss[0, 0]

    iota8 = jnp.arange(_E, dtype=jnp.int32)
    e_all = jnp.concatenate([idx2[:, 0], idx2[:, 1]])
    onehot = (e_all[:, None] == iota8[None, :]).astype(jnp.int32)
    rank = jnp.sum((jnp.cumsum(onehot, axis=0) - onehot) * onehot, axis=1)
    counts = jnp.sum(onehot, axis=0)
    tile_counts = (counts + _BM - 1) // _BM
    end_tile = jnp.cumsum(tile_counts)
    start_row = (end_tile - tile_counts) * _BM
    dest = (start_row[e_all] + rank).astype(jnp.int32)
    tvec = jnp.arange(_TT, dtype=jnp.int32)
    eid = jnp.minimum(
        jnp.sum((tvec[:, None] >= end_tile[None, :]).astype(jnp.int32), axis=1),
        _E - 1)
    eids = jnp.concatenate([eid, end_tile[_E - 1:]]).astype(jnp.int32)
    xs, wrow = _dispatch(dest, x_flat, wa, wb)
    ys = _ffn(eids, xs, W1, b1.reshape(_E, 1, _H), W2, b2.reshape(_E, 1, _D),
              wrow)
    ysh = _shared(x_flat, Ws1, bs1.reshape(1, _H), Ws2, bs2.reshape(1, _D))
    out = _combine(dest[:_S], dest[_S:], ysh, ys)
    return out.reshape(1, _S, _D), router_loss
